# baseline (speedup 1.0000x reference)
import functools

import jax
import jax.numpy as jnp
from jax import lax
from jax.experimental import pallas as pl
from jax.experimental.pallas import tpu as pltpu
from jax.experimental.pallas import tpu_sc as plsc

N = 10000
E = 320000
D_IN = 128
D_HID = 128
D_OUT = 64

NC, NS = 2, 16
EB = 128
NB_W = 80
E_PAD = NC * NS * NB_W * EB
N_ACC = 10240
ROWS_SUB = N_ACC // NS
NBH = NB_W // 2
PAD_ROW = N_ACC - 8

_sc_mesh = plsc.VectorSubcoreMesh(core_axis_name="c", subcore_axis_name="s")


def _hist_body(dst_hbm, out_hbm, dst_v, ones_v, zro_v, acc_sh):
    c = lax.axis_index("c")
    s = lax.axis_index("s")
    pltpu.sync_copy(dst_hbm.at[c, s], dst_v)

    @pl.loop(0, EB)
    def _(i):
        ones_v.at[pl.ds(i, 1), pl.ds(0, 16)][...] = jnp.ones((1, 16), jnp.float32)
        zro_v.at[pl.ds(i, 1), pl.ds(0, 16)][...] = jnp.zeros((1, 16), jnp.float32)

    @pl.loop(0, ROWS_SUB // EB)
    def _(k):
        pltpu.sync_copy(zro_v, acc_sh.at[pl.ds(s * ROWS_SUB + k * EB, EB)])

    plsc.subcore_barrier()

    @pl.loop(0, NB_W)
    def _(b):
        pltpu.sync_copy(ones_v, acc_sh.at[dst_v.at[b]], add=True)

    plsc.subcore_barrier()

    @pl.loop(0, ROWS_SUB // EB)
    def _(k):
        off = s * ROWS_SUB + k * EB
        pltpu.sync_copy(acc_sh.at[pl.ds(off, EB)], out_hbm.at[c, pl.ds(off, EB)])


_hist = functools.partial(
    pl.kernel,
    out_type=jax.ShapeDtypeStruct((NC, N_ACC, 16), jnp.float32),
    mesh=_sc_mesh,
    scratch_types=[
        pltpu.VMEM((NB_W, EB), jnp.int32),
        pltpu.VMEM((EB, 16), jnp.float32),
        pltpu.VMEM((EB, 16), jnp.float32),
        pltpu.VMEM_SHARED((N_ACC, 16), jnp.float32),
    ],
)(_hist_body)


def _msg_body(D, table_hbm, src_hbm, dst_hbm, out_hbm,
              src_v, dst_v, rows0_v, rows1_v, acc_sh, sem0, sem1):
    c = lax.axis_index("c")
    s = lax.axis_index("s")

    @pl.loop(0, EB)
    def _(i):
        @pl.loop(0, D, step=16)
        def _(j):
            rows0_v.at[pl.ds(i, 1), pl.ds(j, 16)][...] = jnp.zeros((1, 16), jnp.float32)

    @pl.loop(0, ROWS_SUB // EB)
    def _(k):
        pltpu.sync_copy(rows0_v, acc_sh.at[pl.ds(s * ROWS_SUB + k * EB, EB)])

    plsc.subcore_barrier()

    @pl.loop(0, 2)
    def _(h):
        pltpu.sync_copy(src_hbm.at[c, s, pl.ds(h * NBH, NBH)], src_v)
        pltpu.sync_copy(dst_hbm.at[c, s, pl.ds(h * NBH, NBH)], dst_v)
        pltpu.make_async_copy(table_hbm.at[src_v.at[0]], rows0_v, sem0).start()

        @pl.loop(0, NBH // 2)
        def _(k):
            b0 = 2 * k
            pltpu.make_async_copy(table_hbm.at[src_v.at[b0 + 1]], rows1_v, sem1).start()
            pltpu.make_async_copy(table_hbm.at[src_v.at[b0]], rows0_v, sem0).wait()
            pltpu.sync_copy(rows0_v, acc_sh.at[dst_v.at[b0]], add=True)

            @pl.when(k < NBH // 2 - 1)
            def _():
                pltpu.make_async_copy(table_hbm.at[src_v.at[b0 + 2]], rows0_v, sem0).start()

            pltpu.make_async_copy(table_hbm.at[src_v.at[b0 + 1]], rows1_v, sem1).wait()
            pltpu.sync_copy(rows1_v, acc_sh.at[dst_v.at[b0 + 1]], add=True)

    plsc.subcore_barrier()

    @pl.loop(0, ROWS_SUB // EB)
    def _(k):
        off = s * ROWS_SUB + k * EB
        pltpu.sync_copy(acc_sh.at[pl.ds(off, EB)], out_hbm.at[c, pl.ds(off, EB)])


def _make_msg(D):
    return functools.partial(
        pl.kernel,
        out_type=jax.ShapeDtypeStruct((NC, N_ACC, D), jnp.float32),
        mesh=_sc_mesh,
        scratch_types=[
            pltpu.VMEM((NBH, EB), jnp.int32),
            pltpu.VMEM((NBH, EB), jnp.int32),
            pltpu.VMEM((EB, D), jnp.float32),
            pltpu.VMEM((EB, D), jnp.float32),
            pltpu.VMEM_SHARED((N_ACC, D), jnp.float32),
            pltpu.SemaphoreType.DMA,
            pltpu.SemaphoreType.DMA,
        ],
    )(functools.partial(_msg_body, D))


_msg128 = _make_msg(D_HID)


_BLK = 1000


def _mm1_kern(x_ref, w_ref, o_ref):
    o_ref[...] = jnp.dot(x_ref[...], w_ref[...],
                         preferred_element_type=jnp.float32,
                         precision=lax.Precision.HIGHEST)


def _mm1(x, W1):
    return pl.pallas_call(
        _mm1_kern,
        grid=(N // _BLK,),
        in_specs=[pl.BlockSpec((_BLK, D_IN), lambda i: (i, 0)),
                  pl.BlockSpec((D_IN, D_HID), lambda i: (0, 0))],
        out_specs=pl.BlockSpec((_BLK, D_HID), lambda i: (i, 0)),
        out_shape=jax.ShapeDtypeStruct((N, D_HID), jnp.float32),
    )(x, W1)


def _dinv_scale_kern(d0_ref, d1_ref, h1_ref, dinv_ref, h1p_ref):
    deg = d0_ref[...][:, :1] + d1_ref[...][:, :1] + 1.0
    dinv = lax.rsqrt(deg)
    dinv_ref[...] = dinv
    h1p_ref[...] = dinv * h1_ref[...]


def _dinv_scale(d0, d1, h1):
    return pl.pallas_call(
        _dinv_scale_kern,
        grid=(N // _BLK,),
        in_specs=[pl.BlockSpec((_BLK, 16), lambda i: (i, 0)),
                  pl.BlockSpec((_BLK, 16), lambda i: (i, 0)),
                  pl.BlockSpec((_BLK, D_HID), lambda i: (i, 0))],
        out_specs=[pl.BlockSpec((_BLK, 1), lambda i: (i, 0)),
                   pl.BlockSpec((_BLK, D_HID), lambda i: (i, 0))],
        out_shape=[jax.ShapeDtypeStruct((N, 1), jnp.float32),
                   jax.ShapeDtypeStruct((N, D_HID), jnp.float32)],
    )(d0, d1, h1)


def _layer1_kern(pa_ref, pb_ref, h1_ref, dinv_ref, b1_ref, w2_ref,
                 h2_ref, h2p_ref):
    dinv = dinv_ref[...]
    h = dinv * (pa_ref[...] + pb_ref[...]) + (dinv * dinv) * h1_ref[...] + b1_ref[...]
    h = jnp.maximum(h, 0.0)
    h2 = jnp.dot(h, w2_ref[...], preferred_element_type=jnp.float32,
                 precision=lax.Precision.HIGHEST)
    h2_ref[...] = h2
    h2p_ref[...] = jnp.concatenate(
        [dinv * h2, jnp.zeros((h2.shape[0], D_HID - D_OUT), jnp.float32)], axis=1)


def _layer1_finish(pa, pb, h1, dinv, b1, W2):
    return pl.pallas_call(
        _layer1_kern,
        grid=(N // _BLK,),
        in_specs=[pl.BlockSpec((_BLK, D_HID), lambda i: (i, 0)),
                  pl.BlockSpec((_BLK, D_HID), lambda i: (i, 0)),
                  pl.BlockSpec((_BLK, D_HID), lambda i: (i, 0)),
                  pl.BlockSpec((_BLK, 1), lambda i: (i, 0)),
                  pl.BlockSpec((1, D_HID), lambda i: (0, 0)),
                  pl.BlockSpec((D_HID, D_OUT), lambda i: (0, 0))],
        out_specs=[pl.BlockSpec((_BLK, D_OUT), lambda i: (i, 0)),
                   pl.BlockSpec((_BLK, D_HID), lambda i: (i, 0))],
        out_shape=[jax.ShapeDtypeStruct((N, D_OUT), jnp.float32),
                   jax.ShapeDtypeStruct((N, D_HID), jnp.float32)],
    )(pa, pb, h1, dinv, b1, W2)


def _final_kern(pa_ref, pb_ref, h2_ref, dinv_ref, b2_ref, z_ref):
    dinv = dinv_ref[...]
    z_ref[...] = (dinv * (pa_ref[...] + pb_ref[...])
                  + (dinv * dinv) * h2_ref[...] + b2_ref[...])


def _final(pa, pb, h2, dinv, b2):
    return pl.pallas_call(
        _final_kern,
        grid=(N // _BLK,),
        in_specs=[pl.BlockSpec((_BLK, D_OUT), lambda i: (i, 0)),
                  pl.BlockSpec((_BLK, D_OUT), lambda i: (i, 0)),
                  pl.BlockSpec((_BLK, D_OUT), lambda i: (i, 0)),
                  pl.BlockSpec((_BLK, 1), lambda i: (i, 0)),
                  pl.BlockSpec((1, D_OUT), lambda i: (0, 0))],
        out_specs=pl.BlockSpec((_BLK, D_OUT), lambda i: (i, 0)),
        out_shape=jax.ShapeDtypeStruct((N, D_OUT), jnp.float32),
    )(pa, pb, h2, dinv, b2)


def kernel(x, edge_index, W1, b1, W2, b2):
    src = edge_index[0].astype(jnp.int32)
    dst = edge_index[1].astype(jnp.int32)
    pad = E_PAD - E
    src_r = jnp.concatenate([src, jnp.zeros((pad,), jnp.int32)]).reshape(NC, NS, NB_W, EB)
    dst_r = jnp.concatenate([dst, jnp.full((pad,), PAD_ROW, jnp.int32)]).reshape(NC, NS, NB_W, EB)

    degp = _hist(dst_r)
    h1 = _mm1(x, W1)
    dinv, h1p = _dinv_scale(degp[0, :N], degp[1, :N], h1)
    acc1 = _msg128(h1p, src_r, dst_r)
    h2, h2p = _layer1_finish(acc1[0, :N], acc1[1, :N], h1, dinv,
                             b1.reshape(1, -1), W2)
    acc2 = _msg128(h2p, src_r, dst_r)
    return _final(acc2[0, :N, :D_OUT], acc2[1, :N, :D_OUT], h2, dinv,
                  b2.reshape(1, -1))


# speedup vs baseline: 1.4511x; 1.4511x over previous
import functools

import jax
import jax.numpy as jnp
from jax import lax
from jax.experimental import pallas as pl
from jax.experimental.pallas import tpu as pltpu
from jax.experimental.pallas import tpu_sc as plsc

N = 10000
E = 320000
D_IN = 128
D_HID = 128
D_OUT = 64

NC, NS = 2, 16
EB = 128
NB_C0 = 53
NB_C1 = 104
NBMAX = max(NB_C0, NB_C1)
E_PAD = NS * (NB_C0 + NB_C1) * EB
N_ACC = 10240
ROWS_SUB = N_ACC // NS
PAD_ROW = N_ACC - 8

_sc_mesh = plsc.VectorSubcoreMesh(core_axis_name="c", subcore_axis_name="s")


def _hist_body(dst_hbm, out_hbm, dst_v, ones_v, zro_v, acc_sh):
    c = lax.axis_index("c")
    s = lax.axis_index("s")
    nb = lax.select(c == 0, NB_C0, NB_C1)
    pltpu.sync_copy(dst_hbm.at[c, s], dst_v)

    @pl.loop(0, EB)
    def _(i):
        ones_v.at[pl.ds(i, 1), pl.ds(0, 16)][...] = jnp.ones((1, 16), jnp.float32)
        zro_v.at[pl.ds(i, 1), pl.ds(0, 16)][...] = jnp.zeros((1, 16), jnp.float32)

    @pl.loop(0, ROWS_SUB // EB)
    def _(k):
        pltpu.sync_copy(zro_v, acc_sh.at[pl.ds(s * ROWS_SUB + k * EB, EB)])

    plsc.subcore_barrier()

    @pl.loop(0, nb)
    def _(b):
        pltpu.sync_copy(ones_v, acc_sh.at[dst_v.at[b]], add=True)

    plsc.subcore_barrier()

    @pl.loop(0, ROWS_SUB // EB)
    def _(k):
        off = s * ROWS_SUB + k * EB
        pltpu.sync_copy(acc_sh.at[pl.ds(off, EB)], out_hbm.at[c, pl.ds(off, EB)])


_hist = functools.partial(
    pl.kernel,
    out_type=jax.ShapeDtypeStruct((NC, N_ACC, 16), jnp.float32),
    mesh=_sc_mesh,
    scratch_types=[
        pltpu.VMEM((NBMAX, EB), jnp.int32),
        pltpu.VMEM((EB, 16), jnp.float32),
        pltpu.VMEM((EB, 16), jnp.float32),
        pltpu.VMEM_SHARED((N_ACC, 16), jnp.float32),
    ],
)(_hist_body)


def _msg_body(D, table_hbm, src_hbm, dst_hbm, out_hbm,
              src_v, dst_v, rows_v, acc_sh, sem):
    c = lax.axis_index("c")
    s = lax.axis_index("s")
    nb = lax.select(c == 0, NB_C0, NB_C1)
    pltpu.sync_copy(src_hbm.at[c, s], src_v)
    pltpu.sync_copy(dst_hbm.at[c, s], dst_v)

    @pl.loop(0, EB)
    def _(i):
        @pl.loop(0, D, step=16)
        def _(j):
            rows_v.at[pl.ds(i, 1), pl.ds(j, 16)][...] = jnp.zeros((1, 16), jnp.float32)

    @pl.loop(0, ROWS_SUB // EB)
    def _(k):
        pltpu.sync_copy(rows_v, acc_sh.at[pl.ds(s * ROWS_SUB + k * EB, EB)])

    plsc.subcore_barrier()

    @pl.loop(0, nb)
    def _(b):
        pltpu.async_copy(table_hbm.at[src_v.at[b]], rows_v, sem).wait()
        pltpu.sync_copy(rows_v, acc_sh.at[dst_v.at[b]], add=True)

    plsc.subcore_barrier()

    @pl.loop(0, ROWS_SUB // EB)
    def _(k):
        off = s * ROWS_SUB + k * EB
        pltpu.sync_copy(acc_sh.at[pl.ds(off, EB)], out_hbm.at[c, pl.ds(off, EB)])


def _make_msg(D):
    return functools.partial(
        pl.kernel,
        out_type=jax.ShapeDtypeStruct((NC, N_ACC, D), jnp.float32),
        mesh=_sc_mesh,
        scratch_types=[
            pltpu.VMEM((NBMAX, EB), jnp.int32),
            pltpu.VMEM((NBMAX, EB), jnp.int32),
            pltpu.VMEM((EB, D), jnp.float32),
            pltpu.VMEM_SHARED((N_ACC, D), jnp.float32),
            pltpu.SemaphoreType.DMA,
        ],
    )(functools.partial(_msg_body, D))


_msg128 = _make_msg(D_HID)


_BLK = 1000


def _mm1_kern(x_ref, w_ref, o_ref):
    o_ref[...] = jnp.dot(x_ref[...], w_ref[...],
                         preferred_element_type=jnp.float32,
                         precision=lax.Precision.HIGHEST)


def _mm1(x, W1):
    return pl.pallas_call(
        _mm1_kern,
        grid=(N // _BLK,),
        in_specs=[pl.BlockSpec((_BLK, D_IN), lambda i: (i, 0)),
                  pl.BlockSpec((D_IN, D_HID), lambda i: (0, 0))],
        out_specs=pl.BlockSpec((_BLK, D_HID), lambda i: (i, 0)),
        out_shape=jax.ShapeDtypeStruct((N, D_HID), jnp.float32),
    )(x, W1)


def _dinv_scale_kern(d0_ref, d1_ref, h1_ref, dinv_ref, h1p_ref):
    deg = d0_ref[...][:, :1] + d1_ref[...][:, :1] + 1.0
    dinv = lax.rsqrt(deg)
    dinv_ref[...] = dinv
    h1p_ref[...] = dinv * h1_ref[...]


def _dinv_scale(d0, d1, h1):
    return pl.pallas_call(
        _dinv_scale_kern,
        grid=(N // _BLK,),
        in_specs=[pl.BlockSpec((_BLK, 16), lambda i: (i, 0)),
                  pl.BlockSpec((_BLK, 16), lambda i: (i, 0)),
                  pl.BlockSpec((_BLK, D_HID), lambda i: (i, 0))],
        out_specs=[pl.BlockSpec((_BLK, 1), lambda i: (i, 0)),
                   pl.BlockSpec((_BLK, D_HID), lambda i: (i, 0))],
        out_shape=[jax.ShapeDtypeStruct((N, 1), jnp.float32),
                   jax.ShapeDtypeStruct((N, D_HID), jnp.float32)],
    )(d0, d1, h1)


def _layer1_kern(pa_ref, pb_ref, h1_ref, dinv_ref, b1_ref, w2_ref,
                 h2_ref, h2p_ref):
    dinv = dinv_ref[...]
    h = dinv * (pa_ref[...] + pb_ref[...]) + (dinv * dinv) * h1_ref[...] + b1_ref[...]
    h = jnp.maximum(h, 0.0)
    h2 = jnp.dot(h, w2_ref[...], preferred_element_type=jnp.float32,
                 precision=lax.Precision.HIGHEST)
    h2_ref[...] = h2
    h2p_ref[...] = jnp.concatenate(
        [dinv * h2, jnp.zeros((h2.shape[0], D_HID - D_OUT), jnp.float32)], axis=1)


def _layer1_finish(pa, pb, h1, dinv, b1, W2):
    return pl.pallas_call(
        _layer1_kern,
        grid=(N // _BLK,),
        in_specs=[pl.BlockSpec((_BLK, D_HID), lambda i: (i, 0)),
                  pl.BlockSpec((_BLK, D_HID), lambda i: (i, 0)),
                  pl.BlockSpec((_BLK, D_HID), lambda i: (i, 0)),
                  pl.BlockSpec((_BLK, 1), lambda i: (i, 0)),
                  pl.BlockSpec((1, D_HID), lambda i: (0, 0)),
                  pl.BlockSpec((D_HID, D_OUT), lambda i: (0, 0))],
        out_specs=[pl.BlockSpec((_BLK, D_OUT), lambda i: (i, 0)),
                   pl.BlockSpec((_BLK, D_HID), lambda i: (i, 0))],
        out_shape=[jax.ShapeDtypeStruct((N, D_OUT), jnp.float32),
                   jax.ShapeDtypeStruct((N, D_HID), jnp.float32)],
    )(pa, pb, h1, dinv, b1, W2)


def _final_kern(pa_ref, pb_ref, h2_ref, dinv_ref, b2_ref, z_ref):
    dinv = dinv_ref[...]
    z_ref[...] = (dinv * (pa_ref[...] + pb_ref[...])
                  + (dinv * dinv) * h2_ref[...] + b2_ref[...])


def _final(pa, pb, h2, dinv, b2):
    return pl.pallas_call(
        _final_kern,
        grid=(N // _BLK,),
        in_specs=[pl.BlockSpec((_BLK, D_OUT), lambda i: (i, 0)),
                  pl.BlockSpec((_BLK, D_OUT), lambda i: (i, 0)),
                  pl.BlockSpec((_BLK, D_OUT), lambda i: (i, 0)),
                  pl.BlockSpec((_BLK, 1), lambda i: (i, 0)),
                  pl.BlockSpec((1, D_OUT), lambda i: (0, 0))],
        out_specs=pl.BlockSpec((_BLK, D_OUT), lambda i: (i, 0)),
        out_shape=jax.ShapeDtypeStruct((N, D_OUT), jnp.float32),
    )(pa, pb, h2, dinv, b2)


def _edge_layout(idx, fill):
    e0 = NS * NB_C0 * EB
    c0 = jnp.full((NS, NBMAX, EB), fill, jnp.int32).at[:, :NB_C0].set(
        idx[:e0].reshape(NS, NB_C0, EB))
    c1 = jnp.concatenate(
        [idx[e0:], jnp.full((NS * NB_C1 * EB - (E - e0),), fill, jnp.int32)]
    ).reshape(NS, NB_C1, EB)
    c1 = jnp.full((NS, NBMAX, EB), fill, jnp.int32).at[:, :NB_C1].set(c1)
    return jnp.stack([c0, c1])


def kernel(x, edge_index, W1, b1, W2, b2):
    src = edge_index[0].astype(jnp.int32)
    dst = edge_index[1].astype(jnp.int32)
    src_r = _edge_layout(src, 0)
    dst_r = _edge_layout(dst, PAD_ROW)

    degp = _hist(dst_r)
    h1 = _mm1(x, W1)
    dinv, h1p = _dinv_scale(degp[0, :N], degp[1, :N], h1)
    acc1 = _msg128(h1p, src_r, dst_r)
    h2, h2p = _layer1_finish(acc1[0, :N], acc1[1, :N], h1, dinv,
                             b1.reshape(1, -1), W2)
    acc2 = _msg128(h2p, src_r, dst_r)
    return _final(acc2[0, :N, :D_OUT], acc2[1, :N, :D_OUT], h2, dinv,
                  b2.reshape(1, -1))


# speedup vs baseline: 1.5010x; 1.0344x over previous
import dataclasses
import functools

import jax
import jax.numpy as jnp
from jax import lax
from jax.experimental import pallas as pl
from jax.experimental.pallas import tpu as pltpu
from jax.experimental.pallas import tpu_sc as plsc

N = 10000
E = 320000
D_IN = 128
D_HID = 128
D_OUT = 64

NC, NS = 2, 16
EB = 128
NB_C0 = 53
NB_C1 = 104
NBMAX = max(NB_C0, NB_C1)
E_PAD = NS * (NB_C0 + NB_C1) * EB
N_ACC = 10240
ROWS_SUB = N_ACC // NS
PAD_ROW = N_ACC - 8
NB_H = 80
E_PAD_H = NC * NS * NB_H * EB

@functools.cache
def _sc_mesh():
    return plsc.VectorSubcoreMesh(core_axis_name="c", subcore_axis_name="s",
                                  num_cores=NC, num_subcores=NS)


def _hist_body(dst_hbm, zrow_hbm, out_hbm, dst_v, hist_t):
    c = lax.axis_index("c")
    s = lax.axis_index("s")
    w = s * NC + c
    pltpu.sync_copy(dst_hbm.at[c, s], dst_v)
    pltpu.sync_copy(zrow_hbm, hist_t)
    ones = jnp.ones((16,), jnp.float32)

    @pl.loop(0, NB_H)
    def _(b):
        @pl.loop(0, EB, step=16)
        def _(j):
            idx = dst_v.at[b, pl.ds(j, 16)][...]
            plsc.addupdate_scatter(hist_t, [idx], ones)

    pltpu.sync_copy(hist_t, out_hbm.at[w])


@functools.cache
def _hist():
    return pl.kernel(
        _hist_body,
        out_type=jax.ShapeDtypeStruct((NC * NS, N_ACC), jnp.float32),
        mesh=_sc_mesh(),
        scratch_types=[
            pltpu.VMEM((NB_H, EB), jnp.int32),
            pltpu.VMEM((N_ACC,), jnp.float32),
        ],
        compiler_params=dataclasses.replace(
            pltpu.CompilerParams(), needs_layout_passes=False),
    )


def _msg_body(D, table_hbm, src_hbm, dst_hbm, zacc_hbm, out_hbm,
              src_v, dst_v, rows_v, acc_sh, sem):
    c = lax.axis_index("c")
    s = lax.axis_index("s")
    pltpu.sync_copy(src_hbm.at[c, s], src_v)
    pltpu.sync_copy(dst_hbm.at[c, s], dst_v)
    pltpu.sync_copy(zacc_hbm, acc_sh.at[pl.ds(s * ROWS_SUB, ROWS_SUB)])

    plsc.subcore_barrier()

    def _gather_scatter(b):
        pltpu.async_copy(table_hbm.at[src_v.at[b]], rows_v, sem).wait()
        pltpu.sync_copy(rows_v, acc_sh.at[dst_v.at[b]], add=True)

    @pl.when(c == 0)
    def _():
        pl.loop(0, NB_C0)(_gather_scatter)

    @pl.when(c != 0)
    def _():
        pl.loop(0, NB_C1)(_gather_scatter)

    plsc.subcore_barrier()

    @pl.loop(0, ROWS_SUB // EB)
    def _(k):
        off = s * ROWS_SUB + k * EB
        pltpu.sync_copy(acc_sh.at[pl.ds(off, EB)], out_hbm.at[c, pl.ds(off, EB)])


@functools.cache
def _make_msg(D):
    return pl.kernel(
        functools.partial(_msg_body, D),
        out_type=jax.ShapeDtypeStruct((NC, N_ACC, D), jnp.float32),
        mesh=_sc_mesh(),
        scratch_types=[
            pltpu.VMEM((NBMAX, EB), jnp.int32),
            pltpu.VMEM((NBMAX, EB), jnp.int32),
            pltpu.VMEM((EB, D), jnp.float32),
            pltpu.VMEM_SHARED((N_ACC, D), jnp.float32),
            pltpu.SemaphoreType.DMA,
        ],
    )


_BLK = 1000


def _mm1_kern(x_ref, w_ref, o_ref):
    o_ref[...] = jnp.dot(x_ref[...], w_ref[...],
                         preferred_element_type=jnp.float32,
                         precision=lax.Precision.HIGHEST)


def _mm1(x, W1):
    return pl.pallas_call(
        _mm1_kern,
        grid=(N // _BLK,),
        in_specs=[pl.BlockSpec((_BLK, D_IN), lambda i: (i, 0)),
                  pl.BlockSpec((D_IN, D_HID), lambda i: (0, 0))],
        out_specs=pl.BlockSpec((_BLK, D_HID), lambda i: (i, 0)),
        out_shape=jax.ShapeDtypeStruct((N, D_HID), jnp.float32),
    )(x, W1)


def _dinv_scale_kern(deg_ref, h1_ref, dinv_ref, h1p_ref):
    dinv = lax.rsqrt(deg_ref[...] + 1.0)
    dinv_ref[...] = dinv
    h1p_ref[...] = dinv * h1_ref[...]


def _dinv_scale(deg, h1):
    return pl.pallas_call(
        _dinv_scale_kern,
        grid=(N // _BLK,),
        in_specs=[pl.BlockSpec((_BLK, 1), lambda i: (i, 0)),
                  pl.BlockSpec((_BLK, D_HID), lambda i: (i, 0))],
        out_specs=[pl.BlockSpec((_BLK, 1), lambda i: (i, 0)),
                   pl.BlockSpec((_BLK, D_HID), lambda i: (i, 0))],
        out_shape=[jax.ShapeDtypeStruct((N, 1), jnp.float32),
                   jax.ShapeDtypeStruct((N, D_HID), jnp.float32)],
    )(deg, h1)


def _layer1_kern(pa_ref, pb_ref, h1_ref, dinv_ref, b1_ref, w2_ref,
                 h2_ref, h2p_ref):
    dinv = dinv_ref[...]
    h = dinv * (pa_ref[...] + pb_ref[...]) + (dinv * dinv) * h1_ref[...] + b1_ref[...]
    h = jnp.maximum(h, 0.0)
    h2 = jnp.dot(h, w2_ref[...], preferred_element_type=jnp.float32,
                 precision=lax.Precision.HIGHEST)
    h2_ref[...] = h2
    h2p_ref[...] = jnp.concatenate(
        [dinv * h2, jnp.zeros((h2.shape[0], D_HID - D_OUT), jnp.float32)], axis=1)


def _layer1_finish(pa, pb, h1, dinv, b1, W2):
    return pl.pallas_call(
        _layer1_kern,
        grid=(N // _BLK,),
        in_specs=[pl.BlockSpec((_BLK, D_HID), lambda i: (i, 0)),
                  pl.BlockSpec((_BLK, D_HID), lambda i: (i, 0)),
                  pl.BlockSpec((_BLK, D_HID), lambda i: (i, 0)),
                  pl.BlockSpec((_BLK, 1), lambda i: (i, 0)),
                  pl.BlockSpec((1, D_HID), lambda i: (0, 0)),
                  pl.BlockSpec((D_HID, D_OUT), lambda i: (0, 0))],
        out_specs=[pl.BlockSpec((_BLK, D_OUT), lambda i: (i, 0)),
                   pl.BlockSpec((_BLK, D_HID), lambda i: (i, 0))],
        out_shape=[jax.ShapeDtypeStruct((N, D_OUT), jnp.float32),
                   jax.ShapeDtypeStruct((N, D_HID), jnp.float32)],
    )(pa, pb, h1, dinv, b1, W2)


def _final_kern(pa_ref, pb_ref, h2_ref, dinv_ref, b2_ref, z_ref):
    dinv = dinv_ref[...]
    z_ref[...] = (dinv * (pa_ref[...] + pb_ref[...])
                  + (dinv * dinv) * h2_ref[...] + b2_ref[...])


def _final(pa, pb, h2, dinv, b2):
    return pl.pallas_call(
        _final_kern,
        grid=(N // _BLK,),
        in_specs=[pl.BlockSpec((_BLK, D_OUT), lambda i: (i, 0)),
                  pl.BlockSpec((_BLK, D_OUT), lambda i: (i, 0)),
                  pl.BlockSpec((_BLK, D_OUT), lambda i: (i, 0)),
                  pl.BlockSpec((_BLK, 1), lambda i: (i, 0)),
                  pl.BlockSpec((1, D_OUT), lambda i: (0, 0))],
        out_specs=pl.BlockSpec((_BLK, D_OUT), lambda i: (i, 0)),
        out_shape=jax.ShapeDtypeStruct((N, D_OUT), jnp.float32),
    )(pa, pb, h2, dinv, b2)


def _edge_layout(idx, fill):
    e0 = NS * NB_C0 * EB
    c0 = jnp.full((NS, NBMAX, EB), fill, jnp.int32).at[:, :NB_C0].set(
        idx[:e0].reshape(NS, NB_C0, EB))
    c1 = jnp.concatenate(
        [idx[e0:], jnp.full((NS * NB_C1 * EB - (E - e0),), fill, jnp.int32)]
    ).reshape(NS, NB_C1, EB)
    c1 = jnp.full((NS, NBMAX, EB), fill, jnp.int32).at[:, :NB_C1].set(c1)
    return jnp.stack([c0, c1])


def kernel(x, edge_index, W1, b1, W2, b2):
    src = edge_index[0].astype(jnp.int32)
    dst = edge_index[1].astype(jnp.int32)
    src_r = _edge_layout(src, 0)
    dst_r = _edge_layout(dst, PAD_ROW)
    dst_bal = jnp.concatenate(
        [dst, jnp.full((E_PAD_H - E,), PAD_ROW, jnp.int32)]
    ).reshape(NC, NS, NB_H, EB)

    zacc = jnp.zeros((ROWS_SUB, D_HID), jnp.float32)
    zrow = jnp.zeros((N_ACC,), jnp.float32)

    histp = _hist()(dst_bal, zrow)
    deg = jnp.sum(histp[:, :N], axis=0)[:, None]
    h1 = _mm1(x, W1)
    dinv, h1p = _dinv_scale(deg, h1)
    acc1 = _make_msg(D_HID)(h1p, src_r, dst_r, zacc)
    h2, h2p = _layer1_finish(acc1[0, :N], acc1[1, :N], h1, dinv,
                             b1.reshape(1, -1), W2)
    acc2 = _make_msg(D_HID)(h2p, src_r, dst_r, zacc)
    return _final(acc2[0, :N, :D_OUT], acc2[1, :N, :D_OUT], h2, dinv,
                  b2.reshape(1, -1))


# speedup vs baseline: 1.7538x; 1.1685x over previous
import dataclasses
import functools

import jax
import jax.numpy as jnp
from jax import lax
from jax.experimental import pallas as pl
from jax.experimental.pallas import tpu as pltpu
from jax.experimental.pallas import tpu_sc as plsc

N = 10000
E = 320000
D_IN = 128
D_HID = 128
D_OUT = 64

NC, NS = 2, 16
EB = 128
NB_C0 = 104
NB_C1 = 53
NBMAX = max(NB_C0, NB_C1)
E_PAD = NS * (NB_C0 + NB_C1) * EB
N_ACC = 10240
ROWS_SUB = N_ACC // NS
PAD_ROW = N_ACC - 8
NB_H = 80
E_PAD_H = NC * NS * NB_H * EB

@functools.cache
def _sc_mesh():
    return plsc.VectorSubcoreMesh(core_axis_name="c", subcore_axis_name="s",
                                  num_cores=NC, num_subcores=NS)


def _hist_body(dst_hbm, zrow_hbm, out_hbm, dst_v, hist_t):
    c = lax.axis_index("c")
    s = lax.axis_index("s")
    w = s * NC + c
    pltpu.sync_copy(dst_hbm.at[c, s], dst_v)
    pltpu.sync_copy(zrow_hbm, hist_t)
    ones = jnp.ones((16,), jnp.float32)

    @pl.loop(0, NB_H)
    def _(b):
        @pl.loop(0, EB, step=16)
        def _(j):
            idx = dst_v.at[b, pl.ds(j, 16)][...]
            plsc.addupdate_scatter(hist_t, [idx], ones)

    pltpu.sync_copy(hist_t, out_hbm.at[w])


@functools.cache
def _hist():
    return pl.kernel(
        _hist_body,
        out_type=jax.ShapeDtypeStruct((NC * NS, N_ACC), jnp.float32),
        mesh=_sc_mesh(),
        scratch_types=[
            pltpu.VMEM((NB_H, EB), jnp.int32),
            pltpu.VMEM((N_ACC,), jnp.float32),
        ],
        compiler_params=dataclasses.replace(
            pltpu.CompilerParams(), needs_layout_passes=False),
    )


def _msg_body(D, table_hbm, src_hbm, dst_hbm, zacc_hbm, out_hbm,
              src_v, dst_v, rows_v, acc_sh, sem):
    c = lax.axis_index("c")
    s = lax.axis_index("s")
    pltpu.sync_copy(src_hbm.at[c, s], src_v)
    pltpu.sync_copy(dst_hbm.at[c, s], dst_v)
    pltpu.sync_copy(zacc_hbm, acc_sh.at[pl.ds(s * ROWS_SUB, ROWS_SUB)])

    plsc.subcore_barrier()

    def _gather_scatter(b):
        pltpu.async_copy(table_hbm.at[src_v.at[b]], rows_v, sem).wait()
        pltpu.sync_copy(rows_v, acc_sh.at[dst_v.at[b]], add=True)

    @pl.when(c == 0)
    def _():
        pl.loop(0, NB_C0)(_gather_scatter)

    @pl.when(c != 0)
    def _():
        pl.loop(0, NB_C1)(_gather_scatter)

    plsc.subcore_barrier()

    @pl.loop(0, ROWS_SUB // EB)
    def _(k):
        off = s * ROWS_SUB + k * EB
        pltpu.sync_copy(acc_sh.at[pl.ds(off, EB)], out_hbm.at[c, pl.ds(off, EB)])


@functools.cache
def _make_msg(D):
    return pl.kernel(
        functools.partial(_msg_body, D),
        out_type=jax.ShapeDtypeStruct((NC, N_ACC, D), jnp.float32),
        mesh=_sc_mesh(),
        scratch_types=[
            pltpu.VMEM((NBMAX, EB), jnp.int32),
            pltpu.VMEM((NBMAX, EB), jnp.int32),
            pltpu.VMEM((EB, D), jnp.float32),
            pltpu.VMEM_SHARED((N_ACC, D), jnp.float32),
            pltpu.SemaphoreType.DMA,
        ],
    )


_BLK = 1000


def _mm1_kern(x_ref, w_ref, o_ref):
    o_ref[...] = jnp.dot(x_ref[...], w_ref[...],
                         preferred_element_type=jnp.float32,
                         precision=lax.Precision.HIGHEST)


def _mm1(x, W1):
    return pl.pallas_call(
        _mm1_kern,
        grid=(N // _BLK,),
        in_specs=[pl.BlockSpec((_BLK, D_IN), lambda i: (i, 0)),
                  pl.BlockSpec((D_IN, D_HID), lambda i: (0, 0))],
        out_specs=pl.BlockSpec((_BLK, D_HID), lambda i: (i, 0)),
        out_shape=jax.ShapeDtypeStruct((N, D_HID), jnp.float32),
    )(x, W1)


def _dinv_scale_kern(deg_ref, h1_ref, dinv_ref, h1p_ref):
    dinv = lax.rsqrt(deg_ref[...] + 1.0)
    dinv_ref[...] = dinv
    h1p_ref[...] = dinv * h1_ref[...]


def _dinv_scale(deg, h1):
    return pl.pallas_call(
        _dinv_scale_kern,
        grid=(N // _BLK,),
        in_specs=[pl.BlockSpec((_BLK, 1), lambda i: (i, 0)),
                  pl.BlockSpec((_BLK, D_HID), lambda i: (i, 0))],
        out_specs=[pl.BlockSpec((_BLK, 1), lambda i: (i, 0)),
                   pl.BlockSpec((_BLK, D_HID), lambda i: (i, 0))],
        out_shape=[jax.ShapeDtypeStruct((N, 1), jnp.float32),
                   jax.ShapeDtypeStruct((N, D_HID), jnp.float32)],
    )(deg, h1)


def _layer1_kern(pa_ref, pb_ref, h1_ref, dinv_ref, b1_ref, w2_ref,
                 h2_ref, h2p_ref):
    dinv = dinv_ref[...]
    h = dinv * (pa_ref[...] + pb_ref[...]) + (dinv * dinv) * h1_ref[...] + b1_ref[...]
    h = jnp.maximum(h, 0.0)
    h2 = jnp.dot(h, w2_ref[...], preferred_element_type=jnp.float32,
                 precision=lax.Precision.HIGHEST)
    h2_ref[...] = h2
    h2p_ref[...] = jnp.concatenate(
        [dinv * h2, jnp.zeros((h2.shape[0], D_HID - D_OUT), jnp.float32)], axis=1)


def _layer1_finish(pa, pb, h1, dinv, b1, W2):
    return pl.pallas_call(
        _layer1_kern,
        grid=(N // _BLK,),
        in_specs=[pl.BlockSpec((_BLK, D_HID), lambda i: (i, 0)),
                  pl.BlockSpec((_BLK, D_HID), lambda i: (i, 0)),
                  pl.BlockSpec((_BLK, D_HID), lambda i: (i, 0)),
                  pl.BlockSpec((_BLK, 1), lambda i: (i, 0)),
                  pl.BlockSpec((1, D_HID), lambda i: (0, 0)),
                  pl.BlockSpec((D_HID, D_OUT), lambda i: (0, 0))],
        out_specs=[pl.BlockSpec((_BLK, D_OUT), lambda i: (i, 0)),
                   pl.BlockSpec((_BLK, D_HID), lambda i: (i, 0))],
        out_shape=[jax.ShapeDtypeStruct((N, D_OUT), jnp.float32),
                   jax.ShapeDtypeStruct((N, D_HID), jnp.float32)],
    )(pa, pb, h1, dinv, b1, W2)


def _final_kern(pa_ref, pb_ref, h2_ref, dinv_ref, b2_ref, z_ref):
    dinv = dinv_ref[...]
    z_ref[...] = (dinv * (pa_ref[...] + pb_ref[...])
                  + (dinv * dinv) * h2_ref[...] + b2_ref[...])


def _final(pa, pb, h2, dinv, b2):
    return pl.pallas_call(
        _final_kern,
        grid=(N // _BLK,),
        in_specs=[pl.BlockSpec((_BLK, D_OUT), lambda i: (i, 0)),
                  pl.BlockSpec((_BLK, D_OUT), lambda i: (i, 0)),
                  pl.BlockSpec((_BLK, D_OUT), lambda i: (i, 0)),
                  pl.BlockSpec((_BLK, 1), lambda i: (i, 0)),
                  pl.BlockSpec((1, D_OUT), lambda i: (0, 0))],
        out_specs=pl.BlockSpec((_BLK, D_OUT), lambda i: (i, 0)),
        out_shape=jax.ShapeDtypeStruct((N, D_OUT), jnp.float32),
    )(pa, pb, h2, dinv, b2)


def _edge_layout(idx, fill):
    e0 = NS * NB_C0 * EB
    c0 = jnp.full((NS, NBMAX, EB), fill, jnp.int32).at[:, :NB_C0].set(
        idx[:e0].reshape(NS, NB_C0, EB))
    c1 = jnp.concatenate(
        [idx[e0:], jnp.full((NS * NB_C1 * EB - (E - e0),), fill, jnp.int32)]
    ).reshape(NS, NB_C1, EB)
    c1 = jnp.full((NS, NBMAX, EB), fill, jnp.int32).at[:, :NB_C1].set(c1)
    return jnp.stack([c0, c1])


def kernel(x, edge_index, W1, b1, W2, b2):
    src = edge_index[0].astype(jnp.int32)
    dst = edge_index[1].astype(jnp.int32)
    src_r = _edge_layout(src, 0)
    dst_r = _edge_layout(dst, PAD_ROW)
    dst_bal = jnp.concatenate(
        [dst, jnp.full((E_PAD_H - E,), PAD_ROW, jnp.int32)]
    ).reshape(NC, NS, NB_H, EB)

    zacc = jnp.zeros((ROWS_SUB, D_HID), jnp.float32)
    zrow = jnp.zeros((N_ACC,), jnp.float32)

    histp = _hist()(dst_bal, zrow)
    deg = jnp.sum(histp[:, :N], axis=0)[:, None]
    h1 = _mm1(x, W1)
    dinv, h1p = _dinv_scale(deg, h1)
    acc1 = _make_msg(D_HID)(h1p, src_r, dst_r, zacc)
    h2, h2p = _layer1_finish(acc1[0, :N], acc1[1, :N], h1, dinv,
                             b1.reshape(1, -1), W2)
    acc2 = _make_msg(D_HID)(h2p, src_r, dst_r, zacc)
    return _final(acc2[0, :N, :D_OUT], acc2[1, :N, :D_OUT], h2, dinv,
                  b2.reshape(1, -1))


# speedup vs baseline: 1.8137x; 1.0341x over previous
import dataclasses
import functools

import jax
import jax.numpy as jnp
from jax import lax
from jax.experimental import pallas as pl
from jax.experimental.pallas import tpu as pltpu
from jax.experimental.pallas import tpu_sc as plsc

N = 10000
E = 320000
D_IN = 128
D_HID = 128
D_OUT = 64

NC, NS = 2, 16
EB = 128
NB_C0 = 97
NB_C1 = 60
NBMAX = max(NB_C0, NB_C1)
E_PAD = NS * (NB_C0 + NB_C1) * EB
N_ACC = 10240
ROWS_SUB = N_ACC // NS
PAD_ROW = N_ACC - 8
NB_H = 80
E_PAD_H = NC * NS * NB_H * EB

@functools.cache
def _sc_mesh():
    return plsc.VectorSubcoreMesh(core_axis_name="c", subcore_axis_name="s",
                                  num_cores=NC, num_subcores=NS)


def _hist_body(dst_hbm, zrow_hbm, out_hbm, dst_v, hist_t):
    c = lax.axis_index("c")
    s = lax.axis_index("s")
    w = s * NC + c
    pltpu.sync_copy(dst_hbm.at[c, s], dst_v)
    pltpu.sync_copy(zrow_hbm, hist_t)
    ones = jnp.ones((16,), jnp.float32)

    @pl.loop(0, NB_H)
    def _(b):
        @pl.loop(0, EB, step=16)
        def _(j):
            idx = dst_v.at[b, pl.ds(j, 16)][...]
            plsc.addupdate_scatter(hist_t, [idx], ones)

    pltpu.sync_copy(hist_t, out_hbm.at[w])


@functools.cache
def _hist():
    return pl.kernel(
        _hist_body,
        out_type=jax.ShapeDtypeStruct((NC * NS, N_ACC), jnp.float32),
        mesh=_sc_mesh(),
        scratch_types=[
            pltpu.VMEM((NB_H, EB), jnp.int32),
            pltpu.VMEM((N_ACC,), jnp.float32),
        ],
        compiler_params=dataclasses.replace(
            pltpu.CompilerParams(), needs_layout_passes=False),
    )


def _msg_body(D, table_hbm, src_hbm, dst_hbm, zacc_hbm, out_hbm,
              src_v, dst_v, rows_v, acc_sh, sem):
    c = lax.axis_index("c")
    s = lax.axis_index("s")
    pltpu.sync_copy(src_hbm.at[c, s], src_v)
    pltpu.sync_copy(dst_hbm.at[c, s], dst_v)
    pltpu.sync_copy(zacc_hbm, acc_sh.at[pl.ds(s * ROWS_SUB, ROWS_SUB)])

    plsc.subcore_barrier()

    def _gather_scatter(b):
        pltpu.async_copy(table_hbm.at[src_v.at[b]], rows_v, sem).wait()
        pltpu.sync_copy(rows_v, acc_sh.at[dst_v.at[b]], add=True)

    @pl.when(c == 0)
    def _():
        pl.loop(0, NB_C0)(_gather_scatter)

    @pl.when(c != 0)
    def _():
        pl.loop(0, NB_C1)(_gather_scatter)

    plsc.subcore_barrier()

    off = s * ROWS_SUB
    pltpu.sync_copy(acc_sh.at[pl.ds(off, ROWS_SUB)], out_hbm.at[c, pl.ds(off, ROWS_SUB)])


@functools.cache
def _make_msg(D):
    return pl.kernel(
        functools.partial(_msg_body, D),
        out_type=jax.ShapeDtypeStruct((NC, N_ACC, D), jnp.float32),
        mesh=_sc_mesh(),
        scratch_types=[
            pltpu.VMEM((NBMAX, EB), jnp.int32),
            pltpu.VMEM((NBMAX, EB), jnp.int32),
            pltpu.VMEM((EB, D), jnp.float32),
            pltpu.VMEM_SHARED((N_ACC, D), jnp.float32),
            pltpu.SemaphoreType.DMA,
        ],
    )


_BLK = 1000


def _mm1_kern(x_ref, w_ref, o_ref):
    o_ref[...] = jnp.dot(x_ref[...], w_ref[...],
                         preferred_element_type=jnp.float32,
                         precision=lax.Precision.HIGHEST)


def _mm1(x, W1):
    return pl.pallas_call(
        _mm1_kern,
        grid=(N // _BLK,),
        in_specs=[pl.BlockSpec((_BLK, D_IN), lambda i: (i, 0)),
                  pl.BlockSpec((D_IN, D_HID), lambda i: (0, 0))],
        out_specs=pl.BlockSpec((_BLK, D_HID), lambda i: (i, 0)),
        out_shape=jax.ShapeDtypeStruct((N, D_HID), jnp.float32),
    )(x, W1)


def _dinv_scale_kern(deg_ref, h1_ref, dinv_ref, h1p_ref):
    dinv = lax.rsqrt(deg_ref[...] + 1.0)
    dinv_ref[...] = dinv
    h1p_ref[...] = dinv * h1_ref[...]


def _dinv_scale(deg, h1):
    return pl.pallas_call(
        _dinv_scale_kern,
        grid=(N // _BLK,),
        in_specs=[pl.BlockSpec((_BLK, 1), lambda i: (i, 0)),
                  pl.BlockSpec((_BLK, D_HID), lambda i: (i, 0))],
        out_specs=[pl.BlockSpec((_BLK, 1), lambda i: (i, 0)),
                   pl.BlockSpec((_BLK, D_HID), lambda i: (i, 0))],
        out_shape=[jax.ShapeDtypeStruct((N, 1), jnp.float32),
                   jax.ShapeDtypeStruct((N, D_HID), jnp.float32)],
    )(deg, h1)


def _layer1_kern(pa_ref, pb_ref, h1_ref, dinv_ref, b1_ref, w2_ref,
                 h2_ref, h2p_ref):
    dinv = dinv_ref[...]
    h = dinv * (pa_ref[...] + pb_ref[...]) + (dinv * dinv) * h1_ref[...] + b1_ref[...]
    h = jnp.maximum(h, 0.0)
    h2 = jnp.dot(h, w2_ref[...], preferred_element_type=jnp.float32,
                 precision=lax.Precision.HIGHEST)
    h2_ref[...] = h2
    h2p_ref[...] = jnp.concatenate(
        [dinv * h2, jnp.zeros((h2.shape[0], D_HID - D_OUT), jnp.float32)], axis=1)


def _layer1_finish(pa, pb, h1, dinv, b1, W2):
    return pl.pallas_call(
        _layer1_kern,
        grid=(N // _BLK,),
        in_specs=[pl.BlockSpec((_BLK, D_HID), lambda i: (i, 0)),
                  pl.BlockSpec((_BLK, D_HID), lambda i: (i, 0)),
                  pl.BlockSpec((_BLK, D_HID), lambda i: (i, 0)),
                  pl.BlockSpec((_BLK, 1), lambda i: (i, 0)),
                  pl.BlockSpec((1, D_HID), lambda i: (0, 0)),
                  pl.BlockSpec((D_HID, D_OUT), lambda i: (0, 0))],
        out_specs=[pl.BlockSpec((_BLK, D_OUT), lambda i: (i, 0)),
                   pl.BlockSpec((_BLK, D_HID), lambda i: (i, 0))],
        out_shape=[jax.ShapeDtypeStruct((N, D_OUT), jnp.float32),
                   jax.ShapeDtypeStruct((N, D_HID), jnp.float32)],
    )(pa, pb, h1, dinv, b1, W2)


def _final_kern(pa_ref, pb_ref, h2_ref, dinv_ref, b2_ref, z_ref):
    dinv = dinv_ref[...]
    z_ref[...] = (dinv * (pa_ref[...] + pb_ref[...])
                  + (dinv * dinv) * h2_ref[...] + b2_ref[...])


def _final(pa, pb, h2, dinv, b2):
    return pl.pallas_call(
        _final_kern,
        grid=(N // _BLK,),
        in_specs=[pl.BlockSpec((_BLK, D_OUT), lambda i: (i, 0)),
                  pl.BlockSpec((_BLK, D_OUT), lambda i: (i, 0)),
                  pl.BlockSpec((_BLK, D_OUT), lambda i: (i, 0)),
                  pl.BlockSpec((_BLK, 1), lambda i: (i, 0)),
                  pl.BlockSpec((1, D_OUT), lambda i: (0, 0))],
        out_specs=pl.BlockSpec((_BLK, D_OUT), lambda i: (i, 0)),
        out_shape=jax.ShapeDtypeStruct((N, D_OUT), jnp.float32),
    )(pa, pb, h2, dinv, b2)


def _edge_layout(idx, fill):
    e0 = NS * NB_C0 * EB
    c0 = jnp.full((NS, NBMAX, EB), fill, jnp.int32).at[:, :NB_C0].set(
        idx[:e0].reshape(NS, NB_C0, EB))
    c1 = jnp.concatenate(
        [idx[e0:], jnp.full((NS * NB_C1 * EB - (E - e0),), fill, jnp.int32)]
    ).reshape(NS, NB_C1, EB)
    c1 = jnp.full((NS, NBMAX, EB), fill, jnp.int32).at[:, :NB_C1].set(c1)
    return jnp.stack([c0, c1])


def kernel(x, edge_index, W1, b1, W2, b2):
    src = edge_index[0].astype(jnp.int32)
    dst = edge_index[1].astype(jnp.int32)
    src_r = _edge_layout(src, 0)
    dst_r = _edge_layout(dst, PAD_ROW)
    dst_bal = jnp.concatenate(
        [dst, jnp.full((E_PAD_H - E,), PAD_ROW, jnp.int32)]
    ).reshape(NC, NS, NB_H, EB)

    zacc = jnp.zeros((ROWS_SUB, D_HID), jnp.float32)
    zrow = jnp.zeros((N_ACC,), jnp.float32)

    histp = _hist()(dst_bal, zrow)
    deg = jnp.sum(histp[:, :N], axis=0)[:, None]
    h1 = _mm1(x, W1)
    dinv, h1p = _dinv_scale(deg, h1)
    acc1 = _make_msg(D_HID)(h1p, src_r, dst_r, zacc)
    h2, h2p = _layer1_finish(acc1[0, :N], acc1[1, :N], h1, dinv,
                             b1.reshape(1, -1), W2)
    acc2 = _make_msg(D_HID)(h2p, src_r, dst_r, zacc)
    return _final(acc2[0, :N, :D_OUT], acc2[1, :N, :D_OUT], h2, dinv,
                  b2.reshape(1, -1))
